# Initial kernel scaffold; baseline (speedup 1.0000x reference)
import functools

import jax
import jax.numpy as jnp
from jax import lax
from jax.experimental import pallas as pl
from jax.experimental.pallas import tpu as pltpu
from jax.experimental.pallas import tpu_sc as plsc

L = 3
_NC = 2
_NS = 16
_NW = _NC * _NS


def _make_sc_agg(N, D, E, chunk):
    ew = E // _NW
    nchunk = ew // chunk
    rpt = N // _NS
    assert ew * _NW == E and nchunk * chunk == ew and rpt * _NS == N
    assert chunk <= 128 and chunk % 8 == 0

    mesh = plsc.VectorSubcoreMesh(core_axis_name="c", subcore_axis_name="s")

    @functools.partial(
        pl.kernel,
        mesh=mesh,
        out_type=jax.ShapeDtypeStruct((_NC, N, D), jnp.float32),
        scratch_types=[
            pltpu.VMEM((chunk,), jnp.int32),
            pltpu.VMEM((chunk,), jnp.int32),
            pltpu.VMEM((chunk, D), jnp.float32),
            pltpu.VMEM_SHARED((N, D), jnp.float32),
            pltpu.SemaphoreType.DMA,
        ],
    )
    def sc_agg(r_hbm, src_hbm, dst_hbm, zero_hbm, out_hbm,
               src_v, dst_v, rows_v, acc_sh, sem):
        c = lax.axis_index("c")
        s = lax.axis_index("s")
        wid = s * _NC + c

        pltpu.sync_copy(zero_hbm, acc_sh.at[pl.ds(s * rpt, rpt)])
        plsc.subcore_barrier()

        def body(i, carry):
            base = wid * ew + i * chunk
            pltpu.sync_copy(src_hbm.at[pl.ds(base, chunk)], src_v)
            pltpu.sync_copy(dst_hbm.at[pl.ds(base, chunk)], dst_v)
            pltpu.async_copy(r_hbm.at[src_v], rows_v, sem).wait()
            pltpu.sync_copy(rows_v, acc_sh.at[dst_v], add=True)
            return carry

        lax.fori_loop(0, nchunk, body, 0)
        plsc.subcore_barrier()
        pltpu.sync_copy(acc_sh.at[pl.ds(s * rpt, rpt)],
                        out_hbm.at[c, pl.ds(s * rpt, rpt)])

    return sc_agg


def _dense_body(h_ref, agg_ref, w1_ref, b1_ref, g1_ref, bt1_ref,
                w2_ref, b2_ref, g2_ref, bt2_ref, eps_ref, o_ref, *, last):
    z = (1.0 + eps_ref[0, 0]) * h_ref[...] + agg_ref[0] + agg_ref[1]
    z = lax.dot_general(z, w1_ref[...], (((1,), (1,)), ((), ())),
                        preferred_element_type=jnp.float32) + b1_ref[...]
    mu = jnp.mean(z, axis=0, keepdims=True)
    var = jnp.mean((z - mu) ** 2, axis=0, keepdims=True)
    z = (z - mu) * lax.rsqrt(var + 1e-5) * g1_ref[...] + bt1_ref[...]
    z = jnp.maximum(z, 0.0)
    z = lax.dot_general(z, w2_ref[...], (((1,), (1,)), ((), ())),
                        preferred_element_type=jnp.float32) + b2_ref[...]
    mu = jnp.mean(z, axis=0, keepdims=True)
    var = jnp.mean((z - mu) ** 2, axis=0, keepdims=True)
    z = (z - mu) * lax.rsqrt(var + 1e-5) * g2_ref[...] + bt2_ref[...]
    if not last:
        z = jnp.maximum(z, 0.0)
    o_ref[...] = z


def _dense(h, agg, w1, b1, g1, bt1, w2, b2, g2, bt2, eps_s, last):
    return pl.pallas_call(
        functools.partial(_dense_body, last=last),
        out_shape=jax.ShapeDtypeStruct(h.shape, jnp.float32),
    )(h, agg, w1, b1.reshape(1, -1), g1.reshape(1, -1), bt1.reshape(1, -1),
      w2, b2.reshape(1, -1), g2.reshape(1, -1), bt2.reshape(1, -1),
      eps_s.reshape(1, 1))


def _relu_body(x_ref, o_ref):
    o_ref[...] = jnp.maximum(x_ref[...], 0.0)


def _relu(x):
    return pl.pallas_call(
        _relu_body, out_shape=jax.ShapeDtypeStruct(x.shape, x.dtype))(x)


def kernel(x, edge_index, W1, b1, g1, bt1, W2, b2, eps, g2, bt2):
    N, D = x.shape
    E = edge_index.shape[1]
    src = edge_index[0]
    dst = edge_index[1]
    zeros = jnp.zeros((N // _NS, D), jnp.float32)

    sc_agg = _make_sc_agg(N, D, E, chunk=80)

    h = x
    r = _relu(x)
    for l in range(L):
        agg = sc_agg(r, src, dst, zeros)
        h = _dense(h, agg, W1[l], b1[l], g1[l], bt1[l],
                   W2[l], b2[l], g2[l], bt2[l], eps[l], last=(l == L - 1))
        r = h
    return h


# speedup vs baseline: 4.7991x; 4.7991x over previous
import functools

import jax
import jax.numpy as jnp
from jax import lax
from jax.experimental import pallas as pl
from jax.experimental.pallas import tpu as pltpu
from jax.experimental.pallas import tpu_sc as plsc

L = 3
_NC = 2
_NS = 16
_NW = _NC * _NS


def _acc_rows(N):
    return -(-N // (_NS * 8)) * _NS * 8


def _make_sc_agg(N, D, E, chunk):
    ew = E // _NW
    nchunk = ew // chunk
    acc_n = _acc_rows(N)
    rpt = acc_n // _NS
    assert ew * _NW == E and nchunk * chunk == ew
    assert chunk <= 128 and chunk % 8 == 0

    mesh = plsc.VectorSubcoreMesh(core_axis_name="c", subcore_axis_name="s")

    @functools.partial(
        pl.kernel,
        mesh=mesh,
        out_type=jax.ShapeDtypeStruct((_NC, acc_n, D), jnp.float32),
        scratch_types=[
            pltpu.VMEM((chunk,), jnp.int32),
            pltpu.VMEM((chunk,), jnp.int32),
            pltpu.VMEM((chunk, D), jnp.float32),
            pltpu.VMEM_SHARED((acc_n, D), jnp.float32),
            pltpu.SemaphoreType.DMA,
        ],
    )
    def sc_agg(r_hbm, src_hbm, dst_hbm, zero_hbm, out_hbm,
               src_v, dst_v, rows_v, acc_sh, sem):
        c = lax.axis_index("c")
        s = lax.axis_index("s")
        wid = s * _NC + c

        pltpu.sync_copy(zero_hbm, acc_sh.at[pl.ds(s * rpt, rpt)])
        plsc.subcore_barrier()

        def body(i, carry):
            base = wid * ew + i * chunk
            pltpu.sync_copy(src_hbm.at[pl.ds(base, chunk)], src_v)
            pltpu.sync_copy(dst_hbm.at[pl.ds(base, chunk)], dst_v)
            pltpu.async_copy(r_hbm.at[src_v], rows_v, sem).wait()
            pltpu.sync_copy(rows_v, acc_sh.at[dst_v], add=True)
            return carry

        lax.fori_loop(0, nchunk, body, 0)
        plsc.subcore_barrier()
        pltpu.sync_copy(acc_sh.at[pl.ds(s * rpt, rpt)],
                        out_hbm.at[c, pl.ds(s * rpt, rpt)])

    return sc_agg


def _dense_body(h_ref, agg_ref, w1_ref, b1_ref, g1_ref, bt1_ref,
                w2_ref, b2_ref, g2_ref, bt2_ref, eps_ref, o_ref, *, last):
    n = h_ref.shape[0]
    z = (1.0 + eps_ref[0, 0]) * h_ref[...] + agg_ref[0, :n] + agg_ref[1, :n]
    z = lax.dot_general(z, w1_ref[...], (((1,), (1,)), ((), ())),
                        preferred_element_type=jnp.float32) + b1_ref[...]
    mu = jnp.mean(z, axis=0, keepdims=True)
    var = jnp.mean((z - mu) ** 2, axis=0, keepdims=True)
    z = (z - mu) * lax.rsqrt(var + 1e-5) * g1_ref[...] + bt1_ref[...]
    z = jnp.maximum(z, 0.0)
    z = lax.dot_general(z, w2_ref[...], (((1,), (1,)), ((), ())),
                        preferred_element_type=jnp.float32) + b2_ref[...]
    mu = jnp.mean(z, axis=0, keepdims=True)
    var = jnp.mean((z - mu) ** 2, axis=0, keepdims=True)
    z = (z - mu) * lax.rsqrt(var + 1e-5) * g2_ref[...] + bt2_ref[...]
    if not last:
        z = jnp.maximum(z, 0.0)
    o_ref[...] = z


def _dense(h, agg, w1, b1, g1, bt1, w2, b2, g2, bt2, eps_s, last):
    return pl.pallas_call(
        functools.partial(_dense_body, last=last),
        out_shape=jax.ShapeDtypeStruct(h.shape, jnp.float32),
    )(h, agg, w1, b1.reshape(1, -1), g1.reshape(1, -1), bt1.reshape(1, -1),
      w2, b2.reshape(1, -1), g2.reshape(1, -1), bt2.reshape(1, -1),
      eps_s.reshape(1, 1))


def _relu_body(x_ref, o_ref):
    o_ref[...] = jnp.maximum(x_ref[...], 0.0)


def _relu(x):
    return pl.pallas_call(
        _relu_body, out_shape=jax.ShapeDtypeStruct(x.shape, x.dtype))(x)


def kernel(x, edge_index, W1, b1, g1, bt1, W2, b2, eps, g2, bt2):
    N, D = x.shape
    E = edge_index.shape[1]
    src = edge_index[0]
    dst = edge_index[1]
    zeros = jnp.zeros((_acc_rows(N) // _NS, D), jnp.float32)

    sc_agg = _make_sc_agg(N, D, E, chunk=80)

    h = x
    r = _relu(x)
    for l in range(L):
        agg = sc_agg(r, src, dst, zeros)
        h = _dense(h, agg, W1[l], b1[l], g1[l], bt1[l],
                   W2[l], b2[l], g2[l], bt2[l], eps[l], last=(l == L - 1))
        r = h
    return h
